# baseline (speedup 1.0000x reference)
import functools

import jax
import jax.numpy as jnp
import numpy as np
from jax import lax
from jax.experimental import pallas as pl
from jax.experimental.pallas import tpu as pltpu
from jax.experimental.pallas import tpu_sc as plsc

B, H, S, D = 8, 16, 4096, 128
SUB = 64
N_CHUNKS = S // SUB
N_SEL = 2048 // SUB
BH = B * H
SCALE = 1.0 / np.sqrt(D)

NC, NS = 2, 16
NW = NC * NS
HPW = BH // NW
NGRP = 8
VROW = SUB * D


def _score_body(q_ref, k_ref, s_ref, cs_ref):
    q = q_ref[0]
    k = k_ref[0]
    s = lax.dot_general(q, k, (((1,), (1,)), ((), ())),
                        preferred_element_type=jnp.float32)
    s = s * SCALE
    s_ref[0] = jnp.zeros((N_CHUNKS, 2 * SUB), jnp.float32)
    for c in range(N_CHUNKS):
        s_ref[0, c:c + 1, 0:SUB] = s[:, c * SUB:(c + 1) * SUB]
    kc = k.reshape(N_CHUNKS, SUB, D)
    kmax = jnp.max(kc, axis=1)
    kmin = jnp.min(kc, axis=1)
    smax = lax.dot_general(q, kmax, (((1,), (1,)), ((), ())),
                           preferred_element_type=jnp.float32)
    smin = lax.dot_general(q, kmin, (((1,), (1,)), ((), ())),
                           preferred_element_type=jnp.float32)
    cs_ref[0] = jnp.maximum(smax, smin)


def _scores(q2, k3, interpret=False):
    return pl.pallas_call(
        _score_body,
        grid=(BH,),
        in_specs=[
            pl.BlockSpec((1, 1, D), lambda i: (i, 0, 0)),
            pl.BlockSpec((1, S, D), lambda i: (i, 0, 0)),
        ],
        out_specs=[
            pl.BlockSpec((1, N_CHUNKS, 2 * SUB), lambda i: (i, 0, 0)),
            pl.BlockSpec((1, 1, N_CHUNKS), lambda i: (i, 0, 0)),
        ],
        out_shape=[
            jax.ShapeDtypeStruct((BH, N_CHUNKS, 2 * SUB), jnp.float32),
            jax.ShapeDtypeStruct((BH, 1, N_CHUNKS), jnp.float32),
        ],
        compiler_params=pltpu.CompilerParams(
            dimension_semantics=("arbitrary",),
        ),
        interpret=interpret,
    )(q2, k3)


_GDN = lax.GatherDimensionNumbers(
    offset_dims=(), collapsed_slice_dims=(0,), start_index_map=(0,))


def _vgather(vec, idx):
    return lax.gather(vec, idx[:, None], _GDN, slice_sizes=(1,),
                      mode=lax.GatherScatterMode.PROMISE_IN_BOUNDS)


def _allmax(v):
    iota16 = lax.iota(jnp.int32, 16)
    for sh in (1, 2, 4, 8):
        v = jnp.maximum(v, _vgather(v, lax.bitwise_xor(iota16, sh)))
    return v


def _allsum(v):
    iota16 = lax.iota(jnp.int32, 16)
    for sh in (1, 2, 4, 8):
        v = v + _vgather(v, lax.bitwise_xor(iota16, sh))
    return v

def _sc_body(cs_hbm, s_hbm, v_hbm, out_hbm,
             cs_v, sel_v, hsel_v, ssel_v, p_v, vbuf_v, out_v, sem, semv):
    wid = lax.axis_index("s") * NC + lax.axis_index("c")

    def head_body(hi, _):
        h = wid * HPW + hi

        pltpu.sync_copy(cs_hbm.at[h], cs_v)

        cvals = [cs_v[pl.ds(16 * t, 16)] for t in range(4)]
        iotas = [lax.iota(jnp.int32, 16) + 16 * t for t in range(4)]

        def rank_body(j, rks):
            jv = jnp.full((16,), j, jnp.int32)
            cj = plsc.load_gather(cs_v, [jv])
            new = []
            for t in range(4):
                gt = jnp.where(cj > cvals[t], 1, 0)
                eq = jnp.where(cj == cvals[t], 1, 0)
                lt = jnp.where(jv < iotas[t], 1, 0)
                new.append(rks[t] + gt + eq * lt)
            return tuple(new)

        zeros4 = tuple(jnp.zeros((16,), jnp.int32) for _ in range(4))
        ranks = lax.fori_loop(0, N_CHUNKS, rank_body, zeros4)

        base = h * N_CHUNKS
        iota16 = lax.iota(jnp.int32, 16)
        for t in range(4):
            gid = iotas[t] + base
            msk = ranks[t] < N_SEL
            plsc.store_scatter(sel_v, [ranks[t]], gid, mask=msk)
            plsc.store_scatter(hsel_v, [2 * ranks[t]], 2 * gid, mask=msk)
            plsc.store_scatter(hsel_v, [2 * ranks[t] + 1], 2 * gid + 1,
                               mask=msk)

        pltpu.async_copy(s_hbm.at[sel_v], ssel_v, sem).wait()

        def max_body(c, m):
            for t in range(4):
                m = jnp.maximum(m, ssel_v[c, pl.ds(16 * t, 16)])
            return m
        macc = lax.fori_loop(0, N_SEL, max_body,
                             jnp.full((16,), -jnp.inf, jnp.float32))
        m = _allmax(macc)

        def exp_body(c, l):
            for t in range(4):
                p = jnp.exp(ssel_v[c, pl.ds(16 * t, 16)] - m)
                p_v[c, pl.ds(16 * t, 16)] = p
                l = l + p
            return l
        lacc = lax.fori_loop(0, N_SEL, exp_body, jnp.zeros((16,), jnp.float32))
        l = _allsum(lacc)

        def _v_copy(g):
            band = lax.bitwise_and(g, 1)
            return pltpu.make_async_copy(
                v_hbm.at[hsel_v.at[pl.ds(g * 8, 8)]],
                vbuf_v.at[pl.ds(band * 8, 8)],
                semv.at[band])

        _v_copy(0).start()

        def group_body(g, accs):
            @pl.when(g + 1 < NGRP)
            def _():
                _v_copy(g + 1).start()
            _v_copy(g).wait()
            base = lax.bitwise_and(g, 1) * 8

            def half_body(j, accs):
                accs = list(accs)
                c = 4 * g + lax.shift_right_logical(j, 1)
                po = lax.bitwise_and(j, 1) * 32
                row = base + j
                cv = jnp.full((16,), c, jnp.int32)
                for t in range(2):
                    for lane in range(16):
                        tk = 16 * t + lane
                        w = plsc.load_gather(
                            p_v, [cv, jnp.full((16,), po + tk, jnp.int32)])
                        bank = lane & 1
                        for u in range(8):
                            v = vbuf_v[row, tk, pl.ds(16 * u, 16)]
                            accs[8 * bank + u] = accs[8 * bank + u] + w * v
                return tuple(accs)

            return lax.fori_loop(0, 8, half_body, accs)

        acc0 = tuple(jnp.zeros((16,), jnp.float32) for _ in range(16))
        accs = lax.fori_loop(0, NGRP, group_body, acc0)

        inv = 1.0 / l
        for u in range(8):
            out_v[pl.ds(16 * u, 16)] = (accs[u] + accs[8 + u]) * inv
        pltpu.sync_copy(out_v, out_hbm.at[h])
        return 0

    lax.fori_loop(0, HPW, head_body, 0)


def _sc_attend(cs, s_rows, v_rows):
    mesh = plsc.VectorSubcoreMesh(core_axis_name="c", subcore_axis_name="s",
                                  num_cores=NC, num_subcores=NS)
    fn = pl.kernel(
        _sc_body,
        out_type=jax.ShapeDtypeStruct((BH, D), jnp.float32),
        mesh=mesh,
        compiler_params=pltpu.CompilerParams(needs_layout_passes=False),
        scratch_types=[
            pltpu.VMEM((N_CHUNKS,), jnp.float32),
            pltpu.VMEM((N_SEL,), jnp.int32),
            pltpu.VMEM((2 * N_SEL,), jnp.int32),
            pltpu.VMEM((N_SEL, 2 * SUB), jnp.float32),
            pltpu.VMEM((N_SEL, SUB), jnp.float32),
            pltpu.VMEM((16, SUB // 2, D), jnp.float32),
            pltpu.VMEM((D,), jnp.float32),
            pltpu.SemaphoreType.DMA,
            pltpu.SemaphoreType.DMA((2,)),
        ],
    )
    return fn(cs, s_rows, v_rows)


def kernel(q, k_cache, v_cache):
    q2 = q.reshape(BH, 1, D)
    k3 = k_cache.reshape(BH, S, D)
    s, cs = _scores(q2, k3)
    cs = cs.reshape(BH, N_CHUNKS)
    s_rows = s.reshape(BH * N_CHUNKS, 2 * SUB)
    v_rows = v_cache.reshape(2 * BH * N_CHUNKS, SUB // 2, D)
    out = _sc_attend(cs, s_rows, v_rows)
    return out.reshape(B, H, D)


# speedup vs baseline: 1.1780x; 1.1780x over previous
import functools

import jax
import jax.numpy as jnp
import numpy as np
from jax import lax
from jax.experimental import pallas as pl
from jax.experimental.pallas import tpu as pltpu
from jax.experimental.pallas import tpu_sc as plsc

B, H, S, D = 8, 16, 4096, 128
SUB = 64
N_CHUNKS = S // SUB
N_SEL = 2048 // SUB
BH = B * H
SCALE = 1.0 / np.sqrt(D)

NC, NS = 2, 16
NW = NC * NS
HPW = BH // NW
NGRP = 8
VROW = SUB * D


def _score_body(q_ref, k_ref, s_ref, cs_ref):
    q = q_ref[0]
    k = k_ref[0]
    s = lax.dot_general(q, k, (((1,), (1,)), ((), ())),
                        preferred_element_type=jnp.float32)
    s = s * SCALE
    s_ref[0] = jnp.zeros((N_CHUNKS, 2 * SUB), jnp.float32)
    for c in range(N_CHUNKS):
        s_ref[0, c:c + 1, 0:SUB] = s[:, c * SUB:(c + 1) * SUB]
    kc = k.reshape(N_CHUNKS, SUB, D)
    kmax = jnp.max(kc, axis=1)
    kmin = jnp.min(kc, axis=1)
    smax = lax.dot_general(q, kmax, (((1,), (1,)), ((), ())),
                           preferred_element_type=jnp.float32)
    smin = lax.dot_general(q, kmin, (((1,), (1,)), ((), ())),
                           preferred_element_type=jnp.float32)
    cs_ref[0] = jnp.maximum(smax, smin)


def _scores(q2, k3, interpret=False):
    return pl.pallas_call(
        _score_body,
        grid=(BH,),
        in_specs=[
            pl.BlockSpec((1, 1, D), lambda i: (i, 0, 0)),
            pl.BlockSpec((1, S, D), lambda i: (i, 0, 0)),
        ],
        out_specs=[
            pl.BlockSpec((1, N_CHUNKS, 2 * SUB), lambda i: (i, 0, 0)),
            pl.BlockSpec((1, 1, N_CHUNKS), lambda i: (i, 0, 0)),
        ],
        out_shape=[
            jax.ShapeDtypeStruct((BH, N_CHUNKS, 2 * SUB), jnp.float32),
            jax.ShapeDtypeStruct((BH, 1, N_CHUNKS), jnp.float32),
        ],
        compiler_params=pltpu.CompilerParams(
            dimension_semantics=("arbitrary",),
        ),
        interpret=interpret,
    )(q2, k3)


_GDN = lax.GatherDimensionNumbers(
    offset_dims=(), collapsed_slice_dims=(0,), start_index_map=(0,))


def _vgather(vec, idx):
    return lax.gather(vec, idx[:, None], _GDN, slice_sizes=(1,),
                      mode=lax.GatherScatterMode.PROMISE_IN_BOUNDS)


def _allmax(v):
    iota16 = lax.iota(jnp.int32, 16)
    for sh in (1, 2, 4, 8):
        v = jnp.maximum(v, _vgather(v, lax.bitwise_xor(iota16, sh)))
    return v


def _allsum(v):
    iota16 = lax.iota(jnp.int32, 16)
    for sh in (1, 2, 4, 8):
        v = v + _vgather(v, lax.bitwise_xor(iota16, sh))
    return v

def _sc_body(cs_hbm, s_hbm, v_hbm, out_hbm,
             cs_v, sel_v, hsel_v, ssel_v, p_v, vbuf_v, psum_v, out_v,
             sem, semv):
    wid = lax.axis_index("s") * NC + lax.axis_index("c")

    def head_body(hi, _):
        h = wid * HPW + hi

        pltpu.sync_copy(cs_hbm.at[h], cs_v)

        cvals = [cs_v[pl.ds(16 * t, 16)] for t in range(4)]
        iotas = [lax.iota(jnp.int32, 16) + 16 * t for t in range(4)]

        def rank_body(j, rks):
            jv = jnp.full((16,), j, jnp.int32)
            cj = plsc.load_gather(cs_v, [jv])
            new = []
            for t in range(4):
                gt = jnp.where(cj > cvals[t], 1, 0)
                eq = jnp.where(cj == cvals[t], 1, 0)
                lt = jnp.where(jv < iotas[t], 1, 0)
                new.append(rks[t] + gt + eq * lt)
            return tuple(new)

        zeros4 = tuple(jnp.zeros((16,), jnp.int32) for _ in range(4))
        ranks = lax.fori_loop(0, N_CHUNKS, rank_body, zeros4)

        base = h * N_CHUNKS
        iota16 = lax.iota(jnp.int32, 16)
        for t in range(4):
            gid = iotas[t] + base
            msk = ranks[t] < N_SEL
            plsc.store_scatter(sel_v, [ranks[t]], gid, mask=msk)
            plsc.store_scatter(hsel_v, [2 * ranks[t]], 2 * gid, mask=msk)
            plsc.store_scatter(hsel_v, [2 * ranks[t] + 1], 2 * gid + 1,
                               mask=msk)

        pltpu.async_copy(s_hbm.at[sel_v], ssel_v, sem).wait()

        def max_body(c, m):
            for t in range(4):
                m = jnp.maximum(m, ssel_v[c, pl.ds(16 * t, 16)])
            return m
        macc = lax.fori_loop(0, N_SEL, max_body,
                             jnp.full((16,), -jnp.inf, jnp.float32))
        m = _allmax(macc)

        def exp_body(c, l):
            for t in range(4):
                p = jnp.exp(ssel_v[c, pl.ds(16 * t, 16)] - m)
                p_v[c, pl.ds(16 * t, 16)] = p
                l = l + p
            return l
        lacc = lax.fori_loop(0, N_SEL, exp_body, jnp.zeros((16,), jnp.float32))
        l = _allsum(lacc)

        def _v_copy(g):
            band = lax.bitwise_and(g, 1)
            return pltpu.make_async_copy(
                v_hbm.at[hsel_v.at[pl.ds(g * 8, 8)]],
                vbuf_v.at[pl.ds(band * 8, 8)],
                semv.at[band])

        _v_copy(0).start()

        zv = jnp.zeros((16,), jnp.float32)

        def z_body(j, _):
            for u in range(8):
                psum_v[j, pl.ds(16 * u, 16)] = zv
            return 0
        lax.fori_loop(0, SUB // 2, z_body, 0)

        def group_body(g, _):
            @pl.when(g + 1 < NGRP)
            def _issue():
                _v_copy(g + 1).start()
            _v_copy(g).wait()
            base = lax.bitwise_and(g, 1) * 8

            def half_body(j, __):
                c = 4 * g + lax.shift_right_logical(j, 1)
                po = lax.bitwise_and(j, 1) * 32
                row = base + j
                cv = jnp.full((16,), c, jnp.int32)

                @plsc.parallel_loop(0, SUB // 2, step=1, unroll=4,
                                    carry=jnp.int32(0))
                def _acc(tk, cr):
                    w = plsc.load_gather(
                        p_v, [cv, jnp.full((16,), po + tk, jnp.int32)])
                    for u in range(8):
                        v = vbuf_v[row, tk, pl.ds(16 * u, 16)]
                        plsc.addupdate(psum_v.at[tk, pl.ds(16 * u, 16)],
                                       w * v)
                    return cr
                return 0

            lax.fori_loop(0, 8, half_body, 0)
            return 0

        lax.fori_loop(0, NGRP, group_body, 0)

        inv = 1.0 / l

        def r_body(j, accs):
            return tuple(accs[u] + psum_v[j, pl.ds(16 * u, 16)]
                         for u in range(8))
        accs = lax.fori_loop(0, SUB // 2, r_body,
                             tuple(zv for _ in range(8)))
        for u in range(8):
            out_v[pl.ds(16 * u, 16)] = accs[u] * inv
        pltpu.sync_copy(out_v, out_hbm.at[h])
        return 0

    lax.fori_loop(0, HPW, head_body, 0)


def _sc_attend(cs, s_rows, v_rows):
    mesh = plsc.VectorSubcoreMesh(core_axis_name="c", subcore_axis_name="s",
                                  num_cores=NC, num_subcores=NS)
    fn = pl.kernel(
        _sc_body,
        out_type=jax.ShapeDtypeStruct((BH, D), jnp.float32),
        mesh=mesh,
        compiler_params=pltpu.CompilerParams(needs_layout_passes=False),
        scratch_types=[
            pltpu.VMEM((N_CHUNKS,), jnp.float32),
            pltpu.VMEM((N_SEL,), jnp.int32),
            pltpu.VMEM((2 * N_SEL,), jnp.int32),
            pltpu.VMEM((N_SEL, 2 * SUB), jnp.float32),
            pltpu.VMEM((N_SEL, SUB), jnp.float32),
            pltpu.VMEM((16, SUB // 2, D), jnp.float32),
            pltpu.VMEM((SUB // 2, D), jnp.float32),
            pltpu.VMEM((D,), jnp.float32),
            pltpu.SemaphoreType.DMA,
            pltpu.SemaphoreType.DMA((2,)),
        ],
    )
    return fn(cs, s_rows, v_rows)


def kernel(q, k_cache, v_cache):
    q2 = q.reshape(BH, 1, D)
    k3 = k_cache.reshape(BH, S, D)
    s, cs = _scores(q2, k3)
    cs = cs.reshape(BH, N_CHUNKS)
    s_rows = s.reshape(BH * N_CHUNKS, 2 * SUB)
    v_rows = v_cache.reshape(2 * BH * N_CHUNKS, SUB // 2, D)
    out = _sc_attend(cs, s_rows, v_rows)
    return out.reshape(B, H, D)


# speedup vs baseline: 1.5193x; 1.2897x over previous
import functools

import jax
import jax.numpy as jnp
import numpy as np
from jax import lax
from jax.experimental import pallas as pl
from jax.experimental.pallas import tpu as pltpu
from jax.experimental.pallas import tpu_sc as plsc

B, H, S, D = 8, 16, 4096, 128
SUB = 64
N_CHUNKS = S // SUB
N_SEL = 2048 // SUB
BH = B * H
SCALE = 1.0 / np.sqrt(D)

NC, NS = 2, 16
NW = NC * NS
HPW = BH // NW
NGRP = 8
VROW = SUB * D


def _score_body(q_ref, k_ref, s_ref, cs_ref):
    q = q_ref[0]
    k = k_ref[0]
    s = lax.dot_general(q, k, (((1,), (1,)), ((), ())),
                        preferred_element_type=jnp.float32)
    s = s * SCALE
    s_ref[0] = jnp.zeros((N_CHUNKS, 2 * SUB), jnp.float32)
    for c in range(N_CHUNKS):
        s_ref[0, c:c + 1, 0:SUB] = s[:, c * SUB:(c + 1) * SUB]
    kc = k.reshape(N_CHUNKS, SUB, D)
    kmax = jnp.max(kc, axis=1)
    kmin = jnp.min(kc, axis=1)
    smax = lax.dot_general(q, kmax, (((1,), (1,)), ((), ())),
                           preferred_element_type=jnp.float32)
    smin = lax.dot_general(q, kmin, (((1,), (1,)), ((), ())),
                           preferred_element_type=jnp.float32)
    cs_ref[0] = jnp.maximum(smax, smin)


def _scores(q2, k3, off, nh, interpret=False):
    return pl.pallas_call(
        _score_body,
        grid=(nh,),
        in_specs=[
            pl.BlockSpec((1, 1, D), lambda i, off=off: (i + off, 0, 0)),
            pl.BlockSpec((1, S, D), lambda i, off=off: (i + off, 0, 0)),
        ],
        out_specs=[
            pl.BlockSpec((1, N_CHUNKS, 2 * SUB), lambda i: (i, 0, 0)),
            pl.BlockSpec((1, 1, N_CHUNKS), lambda i: (i, 0, 0)),
        ],
        out_shape=[
            jax.ShapeDtypeStruct((nh, N_CHUNKS, 2 * SUB), jnp.float32),
            jax.ShapeDtypeStruct((nh, 1, N_CHUNKS), jnp.float32),
        ],
        compiler_params=pltpu.CompilerParams(
            dimension_semantics=("arbitrary",),
        ),
        interpret=interpret,
    )(q2, k3)


_GDN = lax.GatherDimensionNumbers(
    offset_dims=(), collapsed_slice_dims=(0,), start_index_map=(0,))


def _vgather(vec, idx):
    return lax.gather(vec, idx[:, None], _GDN, slice_sizes=(1,),
                      mode=lax.GatherScatterMode.PROMISE_IN_BOUNDS)


def _allmax(v):
    iota16 = lax.iota(jnp.int32, 16)
    for sh in (1, 2, 4, 8):
        v = jnp.maximum(v, _vgather(v, lax.bitwise_xor(iota16, sh)))
    return v


def _allsum(v):
    iota16 = lax.iota(jnp.int32, 16)
    for sh in (1, 2, 4, 8):
        v = v + _vgather(v, lax.bitwise_xor(iota16, sh))
    return v

def _sc_body(hoff, hpw, cs_hbm, s_hbm, v_hbm, out_hbm,
             cs_v, sel_v, hsel_v, ssel_v, p_v, vbuf_v, psum_v, out_v,
             sem, semv):
    wid = lax.axis_index("s") * NC + lax.axis_index("c")

    def head_body(hi, _):
        h = wid * hpw + hi

        pltpu.sync_copy(cs_hbm.at[h], cs_v)

        cvals = [cs_v[pl.ds(16 * t, 16)] for t in range(4)]
        iotas = [lax.iota(jnp.int32, 16) + 16 * t for t in range(4)]

        def rank_body(j, rks):
            jv = jnp.full((16,), j, jnp.int32)
            cj = plsc.load_gather(cs_v, [jv])
            new = []
            for t in range(4):
                gt = jnp.where(cj > cvals[t], 1, 0)
                eq = jnp.where(cj == cvals[t], 1, 0)
                lt = jnp.where(jv < iotas[t], 1, 0)
                new.append(rks[t] + gt + eq * lt)
            return tuple(new)

        zeros4 = tuple(jnp.zeros((16,), jnp.int32) for _ in range(4))
        ranks = lax.fori_loop(0, N_CHUNKS, rank_body, zeros4)

        base = h * N_CHUNKS
        iota16 = lax.iota(jnp.int32, 16)
        for t in range(4):
            gid = iotas[t] + base
            vgid = gid + hoff * N_CHUNKS
            msk = ranks[t] < N_SEL
            plsc.store_scatter(sel_v, [ranks[t]], gid, mask=msk)
            plsc.store_scatter(hsel_v, [2 * ranks[t]], 2 * vgid, mask=msk)
            plsc.store_scatter(hsel_v, [2 * ranks[t] + 1], 2 * vgid + 1,
                               mask=msk)

        pltpu.async_copy(s_hbm.at[sel_v], ssel_v, sem).wait()

        def max_body(c, m):
            for t in range(4):
                m = jnp.maximum(m, ssel_v[c, pl.ds(16 * t, 16)])
            return m
        macc = lax.fori_loop(0, N_SEL, max_body,
                             jnp.full((16,), -jnp.inf, jnp.float32))
        m = _allmax(macc)

        def exp_body(c, l):
            for t in range(4):
                p = jnp.exp(ssel_v[c, pl.ds(16 * t, 16)] - m)
                p_v[c, pl.ds(16 * t, 16)] = p
                l = l + p
            return l
        lacc = lax.fori_loop(0, N_SEL, exp_body, jnp.zeros((16,), jnp.float32))
        l = _allsum(lacc)

        def _v_copy(g):
            band = lax.bitwise_and(g, 1)
            return pltpu.make_async_copy(
                v_hbm.at[hsel_v.at[pl.ds(g * 8, 8)]],
                vbuf_v.at[pl.ds(band * 8, 8)],
                semv.at[band])

        _v_copy(0).start()

        zv = jnp.zeros((16,), jnp.float32)

        def z_body(j, _):
            for u in range(8):
                psum_v[j, pl.ds(16 * u, 16)] = zv
            return 0
        lax.fori_loop(0, SUB // 2, z_body, 0)

        def group_body(g, _):
            @pl.when(g + 1 < NGRP)
            def _issue():
                _v_copy(g + 1).start()
            _v_copy(g).wait()
            base = lax.bitwise_and(g, 1) * 8

            def half_body(j, __):
                c = 4 * g + lax.shift_right_logical(j, 1)
                po = lax.bitwise_and(j, 1) * 32
                row = base + j
                cv = jnp.full((16,), c, jnp.int32)

                @plsc.parallel_loop(0, SUB // 2, step=1, unroll=4,
                                    carry=jnp.int32(0))
                def _acc(tk, cr):
                    w = plsc.load_gather(
                        p_v, [cv, jnp.full((16,), po + tk, jnp.int32)])
                    for u in range(8):
                        v = vbuf_v[row, tk, pl.ds(16 * u, 16)]
                        plsc.addupdate(psum_v.at[tk, pl.ds(16 * u, 16)],
                                       w * v)
                    return cr
                return 0

            lax.fori_loop(0, 8, half_body, 0)
            return 0

        lax.fori_loop(0, NGRP, group_body, 0)

        inv = 1.0 / l

        def r_body(j, accs):
            return tuple(accs[u] + psum_v[j, pl.ds(16 * u, 16)]
                         for u in range(8))
        accs = lax.fori_loop(0, SUB // 2, r_body,
                             tuple(zv for _ in range(8)))
        for u in range(8):
            out_v[pl.ds(16 * u, 16)] = accs[u] * inv
        pltpu.sync_copy(out_v, out_hbm.at[h])
        return 0

    lax.fori_loop(0, hpw, head_body, 0)


def _sc_attend(cs, s_rows, v_rows, hoff, nh):
    mesh = plsc.VectorSubcoreMesh(core_axis_name="c", subcore_axis_name="s",
                                  num_cores=NC, num_subcores=NS)
    fn = pl.kernel(
        functools.partial(_sc_body, hoff, nh // NW),
        out_type=jax.ShapeDtypeStruct((nh, D), jnp.float32),
        mesh=mesh,
        compiler_params=pltpu.CompilerParams(needs_layout_passes=False),
        scratch_types=[
            pltpu.VMEM((N_CHUNKS,), jnp.float32),
            pltpu.VMEM((N_SEL,), jnp.int32),
            pltpu.VMEM((2 * N_SEL,), jnp.int32),
            pltpu.VMEM((N_SEL, 2 * SUB), jnp.float32),
            pltpu.VMEM((N_SEL, SUB), jnp.float32),
            pltpu.VMEM((16, SUB // 2, D), jnp.float32),
            pltpu.VMEM((SUB // 2, D), jnp.float32),
            pltpu.VMEM((D,), jnp.float32),
            pltpu.SemaphoreType.DMA,
            pltpu.SemaphoreType.DMA((2,)),
        ],
    )
    return fn(cs, s_rows, v_rows)


NSLICE = 4


def kernel(q, k_cache, v_cache):
    q2 = q.reshape(BH, 1, D)
    k3 = k_cache.reshape(BH, S, D)
    v_rows = v_cache.reshape(2 * BH * N_CHUNKS, SUB // 2, D)
    nh = BH // NSLICE
    outs = []
    for i in range(NSLICE):
        s, cs = _scores(q2, k3, i * nh, nh)
        cs = cs.reshape(nh, N_CHUNKS)
        s_rows = s.reshape(nh * N_CHUNKS, 2 * SUB)
        outs.append(_sc_attend(cs, s_rows, v_rows, i * nh, nh))
    return jnp.concatenate(outs, axis=0).reshape(B, H, D)
